# Initial kernel scaffold; baseline (speedup 1.0000x reference)
import jax
import jax.numpy as jnp
from jax.experimental import pallas as pl


def kernel(input, game_table):
    raise NotImplementedError("write your pallas kernel here")


# speedup vs baseline: 156.1752x; 156.1752x over previous
import functools

import jax
import jax.numpy as jnp
from jax import lax
from jax.experimental import pallas as pl
from jax.experimental.pallas import tpu as pltpu
from jax.experimental.pallas import tpu_sc as plsc

B = 16384
G = 200
D = 128
V = 5

_NC = 2
_NS = 16
_L = 16
_NW = _NC * _NS
_ROWS_W = B // _NW
_BLK = 128
_NBLK = _ROWS_W // _BLK
_BLK_IN = _BLK * G
_BLK_OUT = _BLK * D
_FULL = G // _L
_TAIL = G - _FULL * _L
_GRP = _L
_NGRP = _BLK // _GRP


@functools.partial(
    pl.kernel,
    mesh=plsc.VectorSubcoreMesh(core_axis_name="c", subcore_axis_name="s"),
    out_type=jax.ShapeDtypeStruct((B * D,), jnp.float32),
    compiler_params=pltpu.CompilerParams(needs_layout_passes=False),
    scratch_types=[
        pltpu.VMEM((_BLK_IN + _L,), jnp.int32),
        pltpu.VMEM((_BLK_OUT,), jnp.float32),
        pltpu.VMEM((V, D), jnp.float32),
        pltpu.VMEM((_GRP * _L,), jnp.int32),
        pltpu.VMEM((V * _GRP,), jnp.float32),
    ],
)
def _sc_embed(inp_hbm, tab_hbm, out_hbm, in_v, out_v, tab_v, acc_v, cnt_v):
    wid = lax.axis_index("s") * _NC + lax.axis_index("c")
    pltpu.sync_copy(tab_hbm, tab_v)
    in_v[pl.ds(_BLK_IN, _L)] = jnp.zeros((_L,), jnp.int32)
    lanes = lax.iota(jnp.int32, _L)
    tailmask = lanes < _TAIL
    lanes16 = lanes * _L
    t = [[tab_v[v, pl.ds(d * _L, _L)] for d in range(D // _L)] for v in range(V)]
    one = jnp.int32(1)

    def count_row(r, _):
        roff = r * G
        acc = jnp.zeros((_L,), jnp.int32)
        for c in range(_FULL):
            x = in_v[pl.ds(roff + c * _L, _L)]
            acc = acc + jnp.where(x < 4, one << ((x & 3) << 3), 0)
        x = in_v[pl.ds(roff + _FULL * _L, _L)]
        acc = acc + jnp.where((x < 4) & tailmask, one << ((x & 3) << 3), 0)
        acc_v[pl.ds((r & (_GRP - 1)) * _L, _L)] = acc
        return 0

    def out_row(r, _):
        obase = r * D
        rg = r & (_GRP - 1)
        rgvec = jnp.broadcast_to(rg, (_L,)).astype(jnp.int32)
        f = [plsc.load_gather(cnt_v, [rgvec + (k * _GRP)]) for k in range(V)]
        for d in range(D // _L):
            o = (f[0] * t[0][d] + f[1] * t[1][d] + f[2] * t[2][d]
                 + f[3] * t[3][d] + f[4] * t[4][d])
            out_v[pl.ds(obase + d * _L, _L)] = o
        return 0

    def do_group(g, _):
        lax.fori_loop(g * _GRP, (g + 1) * _GRP, count_row, 0)
        tot = plsc.load_gather(acc_v, [lanes16])
        for j in range(1, _GRP):
            tot = tot + plsc.load_gather(acc_v, [lanes16 + j])
        c0 = (tot & 255).astype(jnp.float32)
        c1 = ((tot >> 8) & 255).astype(jnp.float32)
        c2 = ((tot >> 16) & 255).astype(jnp.float32)
        c3 = ((tot >> 24) & 255).astype(jnp.float32)
        c4 = jnp.float32(G) - c0 - c1 - c2 - c3
        cnt_v[pl.ds(0 * _GRP, _GRP)] = c0
        cnt_v[pl.ds(1 * _GRP, _GRP)] = c1
        cnt_v[pl.ds(2 * _GRP, _GRP)] = c2
        cnt_v[pl.ds(3 * _GRP, _GRP)] = c3
        cnt_v[pl.ds(4 * _GRP, _GRP)] = c4
        lax.fori_loop(g * _GRP, (g + 1) * _GRP, out_row, 0)
        return 0

    def do_block(blk, _):
        base = (wid * _ROWS_W + blk * _BLK) * G
        pltpu.sync_copy(inp_hbm.at[pl.ds(base, _BLK_IN)],
                        in_v.at[pl.ds(0, _BLK_IN)])
        lax.fori_loop(0, _NGRP, do_group, 0)
        obase = (wid * _ROWS_W + blk * _BLK) * D
        pltpu.sync_copy(out_v, out_hbm.at[pl.ds(obase, _BLK_OUT)])
        return 0

    lax.fori_loop(0, _NBLK, do_block, 0)


def kernel(input, game_table):
    out = _sc_embed(input.reshape(-1), game_table)
    return out.reshape(B, D)


# speedup vs baseline: 167.1699x; 1.0704x over previous
import functools

import jax
import jax.numpy as jnp
from jax import lax
from jax.experimental import pallas as pl
from jax.experimental.pallas import tpu as pltpu
from jax.experimental.pallas import tpu_sc as plsc

B = 16384
G = 200
D = 128
V = 5

_NC = 2
_NS = 16
_L = 16
_NW = _NC * _NS
_ROWS_W = B // _NW
_BLK = 128
_NBLK = _ROWS_W // _BLK
_BLK_IN = _BLK * G
_BLK_OUT = _BLK * D
_FULL = G // _L
_TAIL = G - _FULL * _L
_GRP = _L
_NGRP = _BLK // _GRP


@functools.partial(
    pl.kernel,
    mesh=plsc.VectorSubcoreMesh(core_axis_name="c", subcore_axis_name="s"),
    out_type=jax.ShapeDtypeStruct((B * D,), jnp.float32),
    compiler_params=pltpu.CompilerParams(needs_layout_passes=False),
    scratch_types=[
        pltpu.VMEM((_BLK_IN + _L,), jnp.int32),
        pltpu.VMEM((_BLK_IN + _L,), jnp.int32),
        pltpu.VMEM((_BLK_OUT,), jnp.float32),
        pltpu.VMEM((_BLK_OUT,), jnp.float32),
        pltpu.VMEM((V, D), jnp.float32),
        pltpu.VMEM((_GRP * _L,), jnp.int32),
        pltpu.VMEM((V * _GRP,), jnp.float32),
        pltpu.VMEM((_L,), jnp.int32),
        pltpu.SemaphoreType.DMA,
        pltpu.SemaphoreType.DMA,
        pltpu.SemaphoreType.DMA,
        pltpu.SemaphoreType.DMA,
    ],
)
def _sc_embed(inp_hbm, tab_hbm, out_hbm, in0_v, in1_v, out0_v, out1_v,
              tab_v, acc_v, cnt_v, lut_v, si0, si1, so0, so1):
    wid = lax.axis_index("s") * _NC + lax.axis_index("c")
    pltpu.sync_copy(tab_hbm, tab_v)
    lanes = lax.iota(jnp.int32, _L)
    tailmask = lanes < _TAIL
    lanes16 = lanes * _L
    one = jnp.int32(1)
    lut_v[pl.ds(0, _L)] = jnp.where(lanes < 4, one << ((lanes & 3) << 3), 0)
    in0_v[pl.ds(_BLK_IN, _L)] = jnp.zeros((_L,), jnp.int32)
    in1_v[pl.ds(_BLK_IN, _L)] = jnp.zeros((_L,), jnp.int32)
    t = [[tab_v[v, pl.ds(d * _L, _L)] for d in range(D // _L)] for v in range(V)]

    def compute_block(in_ref, out_ref):
        def count_row(r, _):
            roff = r * G
            acc = jnp.zeros((_L,), jnp.int32)
            for c in range(_FULL):
                x = in_ref[pl.ds(roff + c * _L, _L)]
                acc = acc + plsc.load_gather(lut_v, [x])
            x = in_ref[pl.ds(roff + _FULL * _L, _L)]
            acc = acc + jnp.where(tailmask, plsc.load_gather(lut_v, [x]), 0)
            acc_v[pl.ds((r & (_GRP - 1)) * _L, _L)] = acc
            return 0

        def out_row(r, _):
            obase = r * D
            rg = jnp.broadcast_to(r & (_GRP - 1), (_L,)).astype(jnp.int32)
            f = [plsc.load_gather(cnt_v, [rg + (k * _GRP)]) for k in range(V)]
            for d in range(D // _L):
                o = (f[0] * t[0][d] + f[1] * t[1][d] + f[2] * t[2][d]
                     + f[3] * t[3][d] + f[4] * t[4][d])
                out_ref[pl.ds(obase + d * _L, _L)] = o
            return 0

        def do_group(g, _):
            lax.fori_loop(g * _GRP, (g + 1) * _GRP, count_row, 0)
            tot = plsc.load_gather(acc_v, [lanes16])
            for j in range(1, _GRP):
                tot = tot + plsc.load_gather(acc_v, [lanes16 + j])
            c0 = (tot & 255).astype(jnp.float32)
            c1 = ((tot >> 8) & 255).astype(jnp.float32)
            c2 = ((tot >> 16) & 255).astype(jnp.float32)
            c3 = ((tot >> 24) & 255).astype(jnp.float32)
            c4 = jnp.float32(G) - c0 - c1 - c2 - c3
            cnt_v[pl.ds(0 * _GRP, _GRP)] = c0
            cnt_v[pl.ds(1 * _GRP, _GRP)] = c1
            cnt_v[pl.ds(2 * _GRP, _GRP)] = c2
            cnt_v[pl.ds(3 * _GRP, _GRP)] = c3
            cnt_v[pl.ds(4 * _GRP, _GRP)] = c4
            lax.fori_loop(g * _GRP, (g + 1) * _GRP, out_row, 0)
            return 0

        lax.fori_loop(0, _NGRP, do_group, 0)

    ins = [in0_v, in1_v]
    outs = [out0_v, out1_v]
    sin = [si0, si1]
    sout = [so0, so1]

    def start_in(blk):
        base = (wid * _ROWS_W + blk * _BLK) * G
        return pltpu.async_copy(inp_hbm.at[pl.ds(base, _BLK_IN)],
                                ins[blk % 2].at[pl.ds(0, _BLK_IN)],
                                sin[blk % 2])

    h_in = [None] * _NBLK
    h_out = [None] * _NBLK
    h_in[0] = start_in(0)
    for blk in range(_NBLK):
        if blk + 1 < _NBLK:
            h_in[blk + 1] = start_in(blk + 1)
        h_in[blk].wait()
        if blk >= 2:
            h_out[blk - 2].wait()
        compute_block(ins[blk % 2], outs[blk % 2])
        obase = (wid * _ROWS_W + blk * _BLK) * D
        h_out[blk] = pltpu.async_copy(outs[blk % 2],
                                      out_hbm.at[pl.ds(obase, _BLK_OUT)],
                                      sout[blk % 2])
    h_out[_NBLK - 2].wait()
    h_out[_NBLK - 1].wait()


def kernel(input, game_table):
    out = _sc_embed(input.reshape(-1), game_table)
    return out.reshape(B, D)


# speedup vs baseline: 176.8936x; 1.0582x over previous
import functools

import jax
import jax.numpy as jnp
from jax import lax
from jax.experimental import pallas as pl
from jax.experimental.pallas import tpu as pltpu
from jax.experimental.pallas import tpu_sc as plsc

B = 16384
G = 200
D = 128
V = 5

_NC = 2
_NS = 16
_L = 16
_NW = _NC * _NS
_ROWS_W = B // _NW
_BLK = 128
_NBLK = _ROWS_W // _BLK
_BLK_IN = _BLK * G
_BLK_OUT = _BLK * D
_FULL = G // _L
_TAIL = G - _FULL * _L
_GRP = _L
_NGRP = _BLK // _GRP


@functools.partial(
    pl.kernel,
    mesh=plsc.VectorSubcoreMesh(core_axis_name="c", subcore_axis_name="s"),
    out_type=jax.ShapeDtypeStruct((B * D,), jnp.float32),
    compiler_params=pltpu.CompilerParams(needs_layout_passes=False),
    scratch_types=[
        pltpu.VMEM((_BLK_IN + _L,), jnp.int32),
        pltpu.VMEM((_BLK_IN + _L,), jnp.int32),
        pltpu.VMEM((_BLK_OUT,), jnp.float32),
        pltpu.VMEM((_BLK_OUT,), jnp.float32),
        pltpu.VMEM((V, D), jnp.float32),
        pltpu.VMEM((_BLK * _L,), jnp.int32),
        pltpu.VMEM((_NGRP * V * _GRP,), jnp.float32),
        pltpu.VMEM((_L,), jnp.int32),
        pltpu.SemaphoreType.DMA,
        pltpu.SemaphoreType.DMA,
        pltpu.SemaphoreType.DMA,
        pltpu.SemaphoreType.DMA,
    ],
)
def _sc_embed(inp_hbm, tab_hbm, out_hbm, in0_v, in1_v, out0_v, out1_v,
              tab_v, acc_v, cnt_v, lut_v, si0, si1, so0, so1):
    wid = lax.axis_index("s") * _NC + lax.axis_index("c")
    pltpu.sync_copy(tab_hbm, tab_v)
    lanes = lax.iota(jnp.int32, _L)
    tailmask = lanes < _TAIL
    lanes16 = lanes * _L
    one = jnp.int32(1)
    lut_v[pl.ds(0, _L)] = jnp.where(lanes < 4, one << ((lanes & 3) << 3), 0)
    in0_v[pl.ds(_BLK_IN, _L)] = jnp.zeros((_L,), jnp.int32)
    in1_v[pl.ds(_BLK_IN, _L)] = jnp.zeros((_L,), jnp.int32)
    t = [[tab_v[v, pl.ds(d * _L, _L)] for d in range(D // _L)] for v in range(V)]

    def compute_block(in_ref, out_ref):
        @plsc.parallel_loop(0, _BLK, unroll=2)
        def count_row(r):
            roff = r * G
            acc_a = jnp.zeros((_L,), jnp.int32)
            acc_b = jnp.zeros((_L,), jnp.int32)
            for c in range(0, _FULL, 2):
                acc_a = acc_a + plsc.load_gather(
                    lut_v, [in_ref[pl.ds(roff + c * _L, _L)]])
                acc_b = acc_b + plsc.load_gather(
                    lut_v, [in_ref[pl.ds(roff + (c + 1) * _L, _L)]])
            x = in_ref[pl.ds(roff + _FULL * _L, _L)]
            acc_a = acc_a + jnp.where(tailmask, plsc.load_gather(lut_v, [x]), 0)
            acc_v[pl.ds(r * _L, _L)] = acc_a + acc_b

        @plsc.parallel_loop(0, _NGRP)
        def reduce_group(g):
            base = g * (_GRP * _L)
            tot = plsc.load_gather(acc_v, [lanes16 + base])
            for j in range(1, _GRP):
                tot = tot + plsc.load_gather(acc_v, [lanes16 + (base + j)])
            c0 = (tot & 255).astype(jnp.float32)
            c1 = ((tot >> 8) & 255).astype(jnp.float32)
            c2 = ((tot >> 16) & 255).astype(jnp.float32)
            c3 = ((tot >> 24) & 255).astype(jnp.float32)
            c4 = jnp.float32(G) - c0 - c1 - c2 - c3
            cbase = g * (V * _GRP)
            cnt_v[pl.ds(cbase + 0 * _GRP, _GRP)] = c0
            cnt_v[pl.ds(cbase + 1 * _GRP, _GRP)] = c1
            cnt_v[pl.ds(cbase + 2 * _GRP, _GRP)] = c2
            cnt_v[pl.ds(cbase + 3 * _GRP, _GRP)] = c3
            cnt_v[pl.ds(cbase + 4 * _GRP, _GRP)] = c4

        @plsc.parallel_loop(0, _BLK, unroll=2)
        def out_row(r):
            obase = r * D
            cidx = (r >> 4) * (V * _GRP) + (r & (_GRP - 1))
            cvec = jnp.broadcast_to(cidx, (_L,)).astype(jnp.int32)
            f = [plsc.load_gather(cnt_v, [cvec + (k * _GRP)]) for k in range(V)]
            for d in range(D // _L):
                o = (f[0] * t[0][d] + f[1] * t[1][d] + f[2] * t[2][d]
                     + f[3] * t[3][d] + f[4] * t[4][d])
                out_ref[pl.ds(obase + d * _L, _L)] = o

    ins = [in0_v, in1_v]
    outs = [out0_v, out1_v]
    sin = [si0, si1]
    sout = [so0, so1]

    def start_in(blk):
        base = (wid * _ROWS_W + blk * _BLK) * G
        return pltpu.async_copy(inp_hbm.at[pl.ds(base, _BLK_IN)],
                                ins[blk % 2].at[pl.ds(0, _BLK_IN)],
                                sin[blk % 2])

    h_in = [None] * _NBLK
    h_out = [None] * _NBLK
    h_in[0] = start_in(0)
    for blk in range(_NBLK):
        if blk + 1 < _NBLK:
            h_in[blk + 1] = start_in(blk + 1)
        h_in[blk].wait()
        if blk >= 2:
            h_out[blk - 2].wait()
        compute_block(ins[blk % 2], outs[blk % 2])
        obase = (wid * _ROWS_W + blk * _BLK) * D
        h_out[blk] = pltpu.async_copy(outs[blk % 2],
                                      out_hbm.at[pl.ds(obase, _BLK_OUT)],
                                      sout[blk % 2])
    h_out[_NBLK - 2].wait()
    h_out[_NBLK - 1].wait()


def kernel(input, game_table):
    out = _sc_embed(input.reshape(-1), game_table)
    return out.reshape(B, D)


# speedup vs baseline: 254.6316x; 1.4395x over previous
import functools

import jax
import jax.numpy as jnp
from jax import lax
from jax.experimental import pallas as pl
from jax.experimental.pallas import tpu as pltpu
from jax.experimental.pallas import tpu_sc as plsc

B = 16384
G = 200
D = 128
V = 5

_NC = 2
_NS = 16
_L = 16
_NW = _NC * _NS
_ROWS_W = B // _NW
_BLK = 128
_NBLK = _ROWS_W // _BLK
_FULL = G // _L
_TAIL = G - _FULL * _L
_GRP = _L
_NGRP = _BLK // _GRP


@functools.partial(
    pl.kernel,
    mesh=plsc.VectorSubcoreMesh(core_axis_name="c", subcore_axis_name="s"),
    out_type=jax.ShapeDtypeStruct((B, D), jnp.float32),
    compiler_params=pltpu.CompilerParams(needs_layout_passes=False),
    scratch_types=[
        pltpu.VMEM((_BLK, G), jnp.int32),
        pltpu.VMEM((_BLK, G), jnp.int32),
        pltpu.VMEM((_BLK, D), jnp.float32),
        pltpu.VMEM((_BLK, D), jnp.float32),
        pltpu.VMEM((V, D), jnp.float32),
        pltpu.VMEM((_BLK * _L,), jnp.int32),
        pltpu.VMEM((_NGRP * V * _GRP,), jnp.float32),
        pltpu.VMEM((_L,), jnp.int32),
        pltpu.SemaphoreType.DMA,
        pltpu.SemaphoreType.DMA,
        pltpu.SemaphoreType.DMA,
        pltpu.SemaphoreType.DMA,
    ],
)
def _sc_embed(inp_hbm, tab_hbm, out_hbm, in0_v, in1_v, out0_v, out1_v,
              tab_v, acc_v, cnt_v, lut_v, si0, si1, so0, so1):
    wid = lax.axis_index("s") * _NC + lax.axis_index("c")
    pltpu.sync_copy(tab_hbm, tab_v)
    lanes = lax.iota(jnp.int32, _L)
    tailmask = lanes >= (_L - _TAIL)
    lanes16 = lanes * _L
    one = jnp.int32(1)
    lut_v[pl.ds(0, _L)] = jnp.where(lanes < 4, one << ((lanes & 3) << 3), 0)
    t = [[tab_v[v, pl.ds(d * _L, _L)] for d in range(D // _L)] for v in range(V)]

    def compute_block(in_ref, out_ref):
        @plsc.parallel_loop(0, _BLK, unroll=2)
        def count_row(r):
            acc_a = jnp.zeros((_L,), jnp.int32)
            acc_b = jnp.zeros((_L,), jnp.int32)
            for c in range(0, _FULL, 2):
                acc_a = acc_a + plsc.load_gather(
                    lut_v, [in_ref[r, pl.ds(c * _L, _L)]])
                acc_b = acc_b + plsc.load_gather(
                    lut_v, [in_ref[r, pl.ds((c + 1) * _L, _L)]])
            x = in_ref[r, pl.ds(G - _L, _L)]
            acc_a = acc_a + jnp.where(tailmask, plsc.load_gather(lut_v, [x]), 0)
            acc_v[pl.ds(r * _L, _L)] = acc_a + acc_b

        @plsc.parallel_loop(0, _NGRP)
        def reduce_group(g):
            base = g * (_GRP * _L)
            tot = plsc.load_gather(acc_v, [lanes16 + base])
            for j in range(1, _GRP):
                tot = tot + plsc.load_gather(acc_v, [lanes16 + (base + j)])
            c0 = (tot & 255).astype(jnp.float32)
            c1 = ((tot >> 8) & 255).astype(jnp.float32)
            c2 = ((tot >> 16) & 255).astype(jnp.float32)
            c3 = ((tot >> 24) & 255).astype(jnp.float32)
            c4 = jnp.float32(G) - c0 - c1 - c2 - c3
            cbase = g * (V * _GRP)
            cnt_v[pl.ds(cbase + 0 * _GRP, _GRP)] = c0
            cnt_v[pl.ds(cbase + 1 * _GRP, _GRP)] = c1
            cnt_v[pl.ds(cbase + 2 * _GRP, _GRP)] = c2
            cnt_v[pl.ds(cbase + 3 * _GRP, _GRP)] = c3
            cnt_v[pl.ds(cbase + 4 * _GRP, _GRP)] = c4

        @plsc.parallel_loop(0, _BLK, unroll=2)
        def out_row(r):
            cidx = (r >> 4) * (V * _GRP) + (r & (_GRP - 1))
            cvec = jnp.broadcast_to(cidx, (_L,)).astype(jnp.int32)
            f = [plsc.load_gather(cnt_v, [cvec + (k * _GRP)]) for k in range(V)]
            for d in range(D // _L):
                o = (f[0] * t[0][d] + f[1] * t[1][d] + f[2] * t[2][d]
                     + f[3] * t[3][d] + f[4] * t[4][d])
                out_ref[r, pl.ds(d * _L, _L)] = o

    ins = [in0_v, in1_v]
    outs = [out0_v, out1_v]
    sin = [si0, si1]
    sout = [so0, so1]

    def start_in(blk):
        rowbase = wid * _ROWS_W + blk * _BLK
        return pltpu.async_copy(inp_hbm.at[pl.ds(rowbase, _BLK), :],
                                ins[blk % 2], sin[blk % 2])

    h_in = [None] * _NBLK
    h_out = [None] * _NBLK
    h_in[0] = start_in(0)
    for blk in range(_NBLK):
        if blk + 1 < _NBLK:
            h_in[blk + 1] = start_in(blk + 1)
        h_in[blk].wait()
        if blk >= 2:
            h_out[blk - 2].wait()
        compute_block(ins[blk % 2], outs[blk % 2])
        rowbase = wid * _ROWS_W + blk * _BLK
        h_out[blk] = pltpu.async_copy(outs[blk % 2],
                                      out_hbm.at[pl.ds(rowbase, _BLK), :],
                                      sout[blk % 2])
    h_out[_NBLK - 2].wait()
    h_out[_NBLK - 1].wait()


def kernel(input, game_table):
    return _sc_embed(input, game_table)
